# Initial kernel scaffold; baseline (speedup 1.0000x reference)
import jax
import jax.numpy as jnp
from jax.experimental import pallas as pl


def kernel(x, edge_index, W, a):
    raise NotImplementedError("write your pallas kernel here")


# speedup vs baseline: 20.6444x; 20.6444x over previous
import jax
import jax.numpy as jnp
from jax import lax
from jax.experimental import pallas as pl
from jax.experimental.pallas import tpu as pltpu
from jax.experimental.pallas import tpu_sc as plsc

N_NODES = 10000
N_EDGES = 320000
D = 128

NC = 2
NS = 16
NWORK = NC * NS
EDGES_PER_WORKER = N_EDGES // NWORK
BATCH = 80
NBATCH = EDGES_PER_WORKER // BATCH
N_PAD = 10240
ROWS_PER_TILE = N_PAD // NS
ZCHUNK = 128
NCHUNK = ROWS_PER_TILE // ZCHUNK

R_BLK = 2000
N_BLKS = N_NODES // R_BLK


def _prep_body(x_ref, w_ref, a1_ref, y_ref, p_ref, cmax_ref):
    pidx = pl.program_id(0)
    blk = pl.program_id(1)

    @pl.when((pidx == 0) & (blk == 0))
    def _():
        cmax_ref[0] = -jnp.inf

    z = jnp.dot(x_ref[...], w_ref[...], preferred_element_type=jnp.float32)
    s1 = jnp.dot(z, a1_ref[...], preferred_element_type=jnp.float32)

    @pl.when(pidx == 0)
    def _():
        cmax_ref[0] = jnp.maximum(cmax_ref[0], jnp.max(s1))

    p = jnp.exp(s1 - cmax_ref[0])
    y_ref[...] = p * z
    p_ref[...] = p


def _prep(x, W, a1):
    return pl.pallas_call(
        _prep_body,
        grid=(2, N_BLKS),
        in_specs=[
            pl.BlockSpec((R_BLK, D), lambda p, i: (i, 0)),
            pl.BlockSpec((D, D), lambda p, i: (0, 0)),
            pl.BlockSpec((D, 1), lambda p, i: (0, 0)),
        ],
        out_specs=[
            pl.BlockSpec((R_BLK, D), lambda p, i: (i, 0)),
            pl.BlockSpec((R_BLK, 1), lambda p, i: (i, 0)),
        ],
        out_shape=[
            jax.ShapeDtypeStruct((N_NODES, D), jnp.float32),
            jax.ShapeDtypeStruct((N_NODES, 1), jnp.float32),
        ],
        scratch_shapes=[pltpu.SMEM((1,), jnp.float32)],
    )(x, W, a1)


def _sc_body(y_hbm, p_hbm, src_hbm, dst_hbm, zrows_hbm, zvec_hbm,
             out_hbm, dout_hbm,
             src_b, dst_b, rows, pv, zbuf, dzb, h_sh, d_sh, sem, sem2):
    c = lax.axis_index("c")
    s = lax.axis_index("s")
    base = (c * NS + s) * EDGES_PER_WORKER

    pltpu.sync_copy(zrows_hbm, zbuf)
    for j in range(NCHUNK):
        pltpu.sync_copy(zbuf, h_sh.at[pl.ds(s * ROWS_PER_TILE + j * ZCHUNK, ZCHUNK), :])
    pltpu.sync_copy(zvec_hbm, dzb)
    pltpu.sync_copy(dzb, d_sh.at[pl.ds(s * ROWS_PER_TILE, ROWS_PER_TILE)])
    plsc.subcore_barrier()

    def batch(b, carry):
        off = base + b * BATCH
        pltpu.sync_copy(src_hbm.at[pl.ds(off, BATCH)], src_b)
        pltpu.sync_copy(dst_hbm.at[pl.ds(off, BATCH)], dst_b)
        cp1 = pltpu.async_copy(y_hbm.at[src_b], rows, sem)
        cp2 = pltpu.async_copy(p_hbm.at[src_b], pv, sem2)
        cp1.wait()
        cp2.wait()
        pltpu.sync_copy(rows, h_sh.at[dst_b], add=True)
        pltpu.sync_copy(pv, d_sh.at[dst_b], add=True)
        return carry

    lax.fori_loop(0, NBATCH, batch, 0)
    plsc.subcore_barrier()

    for j in range(NCHUNK):
        r0 = s * ROWS_PER_TILE + j * ZCHUNK
        pltpu.sync_copy(h_sh.at[pl.ds(r0, ZCHUNK), :], zbuf)
        pltpu.sync_copy(zbuf, out_hbm.at[c, pl.ds(r0, ZCHUNK), :])
    pltpu.sync_copy(d_sh.at[pl.ds(s * ROWS_PER_TILE, ROWS_PER_TILE)], dzb)
    pltpu.sync_copy(dzb, dout_hbm.at[c, pl.ds(s * ROWS_PER_TILE, ROWS_PER_TILE)])


def _sc_scatter(y, p, src, dst):
    mesh = plsc.VectorSubcoreMesh(core_axis_name="c", subcore_axis_name="s")
    zrows = jnp.zeros((ZCHUNK, D), jnp.float32)
    zvec = jnp.zeros((ROWS_PER_TILE,), jnp.float32)
    kern = pl.kernel(
        _sc_body,
        out_type=[
            jax.ShapeDtypeStruct((NC, N_PAD, D), jnp.float32),
            jax.ShapeDtypeStruct((NC, N_PAD), jnp.float32),
        ],
        mesh=mesh,
        scratch_types=[
            pltpu.VMEM((BATCH,), jnp.int32),
            pltpu.VMEM((BATCH,), jnp.int32),
            pltpu.VMEM((BATCH, D), jnp.float32),
            pltpu.VMEM((BATCH,), jnp.float32),
            pltpu.VMEM((ZCHUNK, D), jnp.float32),
            pltpu.VMEM((ROWS_PER_TILE,), jnp.float32),
            pltpu.VMEM_SHARED((N_PAD, D), jnp.float32),
            pltpu.VMEM_SHARED((N_PAD,), jnp.float32),
            pltpu.SemaphoreType.DMA,
            pltpu.SemaphoreType.DMA,
        ],
    )
    return kern(y, p, src, dst, zrows, zvec)


def _finish_body(hp_ref, dp_ref, out_ref):
    hp = hp_ref[...]
    dp = dp_ref[...]
    tot = hp[0] + hp[1]
    den = dp[0] + dp[1]
    inv = jnp.where(den > 0.0, 1.0 / jnp.where(den > 0.0, den, 1.0), 0.0)
    out_ref[...] = tot * inv


def _finish(h_parts, d_parts):
    return pl.pallas_call(
        _finish_body,
        grid=(N_BLKS,),
        in_specs=[
            pl.BlockSpec((NC, R_BLK, D), lambda i: (0, i, 0)),
            pl.BlockSpec((NC, R_BLK, 1), lambda i: (0, i, 0)),
        ],
        out_specs=pl.BlockSpec((R_BLK, D), lambda i: (i, 0)),
        out_shape=jax.ShapeDtypeStruct((N_NODES, D), jnp.float32),
    )(h_parts, d_parts)


@jax.jit
def kernel(x, edge_index, W, a):
    ei = edge_index.astype(jnp.int32)
    src = ei[0]
    dst = ei[1]
    a1 = a[:D].reshape(D, 1)
    y, p2d = _prep(x, W, a1)
    p = p2d.reshape(N_NODES)
    h_parts, d_parts = _sc_scatter(y, p, src, dst)
    return _finish(h_parts[:, :N_NODES], d_parts[:, :N_NODES, None])


# speedup vs baseline: 30.3893x; 1.4720x over previous
import jax
import jax.numpy as jnp
from jax import lax
from jax.experimental import pallas as pl
from jax.experimental.pallas import tpu as pltpu
from jax.experimental.pallas import tpu_sc as plsc

N_NODES = 10000
N_EDGES = 320000
D = 128

NC = 2
NS = 16
NWORK = NC * NS
EDGES_PER_WORKER = N_EDGES // NWORK
BATCH = 80
NBATCH = EDGES_PER_WORKER // BATCH
N_PAD = 10240
ROWS_PER_TILE = N_PAD // NS
ZCHUNK = 128
NCHUNK = ROWS_PER_TILE // ZCHUNK

R_BLK = 2000
N_BLKS = N_NODES // R_BLK


def _prep_body(x_ref, w_ref, a1_ref, y_ref, p_ref, cmax_ref):
    pidx = pl.program_id(0)
    blk = pl.program_id(1)

    @pl.when((pidx == 0) & (blk == 0))
    def _():
        cmax_ref[0] = -jnp.inf

    z = jnp.dot(x_ref[...], w_ref[...], preferred_element_type=jnp.float32)
    s1 = jnp.dot(z, a1_ref[...], preferred_element_type=jnp.float32)

    @pl.when(pidx == 0)
    def _():
        cmax_ref[0] = jnp.maximum(cmax_ref[0], jnp.max(s1))

    p = jnp.exp(s1 - cmax_ref[0])
    y_ref[...] = p * z
    p_ref[...] = p


def _prep(x, W, a1):
    return pl.pallas_call(
        _prep_body,
        grid=(2, N_BLKS),
        in_specs=[
            pl.BlockSpec((R_BLK, D), lambda p, i: (i, 0)),
            pl.BlockSpec((D, D), lambda p, i: (0, 0)),
            pl.BlockSpec((D, 1), lambda p, i: (0, 0)),
        ],
        out_specs=[
            pl.BlockSpec((R_BLK, D), lambda p, i: (i, 0)),
            pl.BlockSpec((R_BLK, 1), lambda p, i: (i, 0)),
        ],
        out_shape=[
            jax.ShapeDtypeStruct((N_NODES, D), jnp.float32),
            jax.ShapeDtypeStruct((N_NODES, 1), jnp.float32),
        ],
        scratch_shapes=[pltpu.SMEM((1,), jnp.float32)],
    )(x, W, a1)


def _sc_body(y_hbm, p_hbm, src_hbm, dst_hbm, zrows_hbm, zvec_hbm,
             out_hbm, dout_hbm,
             src0, src1, dst0, dst1, rows0, rows1, pv0, pv1, zbuf, dzb,
             h_sh, d_sh, semA0, semA1, semB0, semB1):
    c = lax.axis_index("c")
    s = lax.axis_index("s")
    base = (c * NS + s) * EDGES_PER_WORKER

    pltpu.sync_copy(zrows_hbm, zbuf)
    for j in range(NCHUNK):
        pltpu.sync_copy(zbuf, h_sh.at[pl.ds(s * ROWS_PER_TILE + j * ZCHUNK, ZCHUNK), :])
    pltpu.sync_copy(zvec_hbm, dzb)
    pltpu.sync_copy(dzb, d_sh.at[pl.ds(s * ROWS_PER_TILE, ROWS_PER_TILE)])
    plsc.subcore_barrier()

    def issue(b, src_buf, dst_buf, rows_buf, pv_buf, sA, sB):
        off = base + b * BATCH
        pltpu.sync_copy(src_hbm.at[pl.ds(off, BATCH)], src_buf)
        pltpu.sync_copy(dst_hbm.at[pl.ds(off, BATCH)], dst_buf)
        cp1 = pltpu.async_copy(y_hbm.at[src_buf], rows_buf, sA)
        cp2 = pltpu.async_copy(p_hbm.at[src_buf], pv_buf, sB)
        return cp1, cp2

    def scatter(dst_buf, rows_buf, pv_buf):
        pltpu.sync_copy(rows_buf, h_sh.at[dst_buf], add=True)
        pltpu.sync_copy(pv_buf, d_sh.at[dst_buf], add=True)

    issue(0, src0, dst0, rows0, pv0, semA0, semB0)

    def pair(i, carry):
        b0 = 2 * i
        d1a, d1b = issue(b0 + 1, src1, dst1, rows1, pv1, semA1, semB1)
        pltpu.make_async_copy(y_hbm.at[src0], rows0, semA0).wait()
        pltpu.make_async_copy(p_hbm.at[src0], pv0, semB0).wait()
        scatter(dst0, rows0, pv0)

        @pl.when(b0 + 2 < NBATCH)
        def _():
            issue(b0 + 2, src0, dst0, rows0, pv0, semA0, semB0)

        d1a.wait()
        d1b.wait()
        scatter(dst1, rows1, pv1)
        return carry

    lax.fori_loop(0, NBATCH // 2, pair, 0)

    pltpu.make_async_copy(y_hbm.at[src0], rows0, semA0).wait()
    pltpu.make_async_copy(p_hbm.at[src0], pv0, semB0).wait()
    scatter(dst0, rows0, pv0)
    plsc.subcore_barrier()

    for j in range(NCHUNK):
        r0 = s * ROWS_PER_TILE + j * ZCHUNK
        pltpu.sync_copy(h_sh.at[pl.ds(r0, ZCHUNK), :], zbuf)
        pltpu.sync_copy(zbuf, out_hbm.at[c, pl.ds(r0, ZCHUNK), :])
    pltpu.sync_copy(d_sh.at[pl.ds(s * ROWS_PER_TILE, ROWS_PER_TILE)], dzb)
    pltpu.sync_copy(dzb, dout_hbm.at[c, pl.ds(s * ROWS_PER_TILE, ROWS_PER_TILE)])


def _sc_scatter(y, p, src, dst):
    mesh = plsc.VectorSubcoreMesh(core_axis_name="c", subcore_axis_name="s")
    zrows = jnp.zeros((ZCHUNK, D), jnp.float32)
    zvec = jnp.zeros((ROWS_PER_TILE,), jnp.float32)
    kern = pl.kernel(
        _sc_body,
        out_type=[
            jax.ShapeDtypeStruct((NC, N_PAD, D), jnp.float32),
            jax.ShapeDtypeStruct((NC, N_PAD), jnp.float32),
        ],
        mesh=mesh,
        scratch_types=[
            pltpu.VMEM((BATCH,), jnp.int32),
            pltpu.VMEM((BATCH,), jnp.int32),
            pltpu.VMEM((BATCH,), jnp.int32),
            pltpu.VMEM((BATCH,), jnp.int32),
            pltpu.VMEM((BATCH, D), jnp.float32),
            pltpu.VMEM((BATCH, D), jnp.float32),
            pltpu.VMEM((BATCH,), jnp.float32),
            pltpu.VMEM((BATCH,), jnp.float32),
            pltpu.VMEM((ZCHUNK, D), jnp.float32),
            pltpu.VMEM((ROWS_PER_TILE,), jnp.float32),
            pltpu.VMEM_SHARED((N_PAD, D), jnp.float32),
            pltpu.VMEM_SHARED((N_PAD,), jnp.float32),
            pltpu.SemaphoreType.DMA,
            pltpu.SemaphoreType.DMA,
            pltpu.SemaphoreType.DMA,
            pltpu.SemaphoreType.DMA,
        ],
    )
    return kern(y, p, src, dst, zrows, zvec)


def _finish_body(hp_ref, dp_ref, out_ref):
    hp = hp_ref[...]
    dp = dp_ref[...]
    tot = hp[0] + hp[1]
    den = dp[0] + dp[1]
    inv = jnp.where(den > 0.0, 1.0 / jnp.where(den > 0.0, den, 1.0), 0.0)
    out_ref[...] = tot * inv


def _finish(h_parts, d_parts):
    return pl.pallas_call(
        _finish_body,
        grid=(N_BLKS,),
        in_specs=[
            pl.BlockSpec((NC, R_BLK, D), lambda i: (0, i, 0)),
            pl.BlockSpec((NC, R_BLK, 1), lambda i: (0, i, 0)),
        ],
        out_specs=pl.BlockSpec((R_BLK, D), lambda i: (i, 0)),
        out_shape=jax.ShapeDtypeStruct((N_NODES, D), jnp.float32),
    )(h_parts, d_parts)


@jax.jit
def kernel(x, edge_index, W, a):
    ei = edge_index.astype(jnp.int32)
    src = ei[0]
    dst = ei[1]
    a1 = a[:D].reshape(D, 1)
    y, p2d = _prep(x, W, a1)
    p = p2d.reshape(N_NODES)
    h_parts, d_parts = _sc_scatter(y, p, src, dst)
    return _finish(h_parts[:, :N_NODES], d_parts[:, :N_NODES, None])


# speedup vs baseline: 31.1367x; 1.0246x over previous
import jax
import jax.numpy as jnp
from jax import lax
from jax.experimental import pallas as pl
from jax.experimental.pallas import tpu as pltpu
from jax.experimental.pallas import tpu_sc as plsc

N_NODES = 10000
N_EDGES = 320000
D = 128

NC = 2
NS = 16
NWORK = NC * NS
EDGES_PER_WORKER = N_EDGES // NWORK
BATCH = 80
NBATCH = EDGES_PER_WORKER // BATCH
N_PAD_H = 10112
RPT_H = N_PAD_H // NS
H_CHUNKS = (128, 128, 128, 128, 120)
N_PAD_D = 10240
RPT_D = N_PAD_D // NS

R_BLK = 2000
N_BLKS = N_NODES // R_BLK


def _prep_body(x_ref, w_ref, a1_ref, y_ref, p_ref, cmax_ref):
    pidx = pl.program_id(0)
    blk = pl.program_id(1)

    @pl.when((pidx == 0) & (blk == 0))
    def _():
        cmax_ref[0] = -jnp.inf

    z = jnp.dot(x_ref[...], w_ref[...], preferred_element_type=jnp.float32)
    s1 = jnp.dot(z, a1_ref[...], preferred_element_type=jnp.float32)

    @pl.when(pidx == 0)
    def _():
        cmax_ref[0] = jnp.maximum(cmax_ref[0], jnp.max(s1))

    p = jnp.exp(s1 - cmax_ref[0])
    y_ref[...] = p * z
    p_ref[...] = p


def _prep(x, W, a1):
    return pl.pallas_call(
        _prep_body,
        grid=(2, N_BLKS),
        in_specs=[
            pl.BlockSpec((R_BLK, D), lambda p, i: (i, 0)),
            pl.BlockSpec((D, D), lambda p, i: (0, 0)),
            pl.BlockSpec((D, 1), lambda p, i: (0, 0)),
        ],
        out_specs=[
            pl.BlockSpec((R_BLK, D), lambda p, i: (i, 0)),
            pl.BlockSpec((R_BLK, 1), lambda p, i: (i, 0)),
        ],
        out_shape=[
            jax.ShapeDtypeStruct((N_NODES, D), jnp.float32),
            jax.ShapeDtypeStruct((N_NODES, 1), jnp.float32),
        ],
        scratch_shapes=[pltpu.SMEM((1,), jnp.float32)],
    )(x, W, a1)


def _sc_body(y_hbm, p_hbm, src_hbm, dst_hbm, zrows_hbm, zvec_hbm,
             out_hbm, dout_hbm,
             src0, dst0, rows0, pv0, src1, dst1, rows1, pv1,
             src2, dst2, rows2, pv2, zbuf, dzb,
             h_sh, d_sh, semA0, semB0, semA1, semB1, semA2, semB2):
    c = lax.axis_index("c")
    s = lax.axis_index("s")
    base = (c * NS + s) * EDGES_PER_WORKER

    pltpu.sync_copy(zrows_hbm, zbuf)
    r0 = s * RPT_H
    for sz in H_CHUNKS:
        pltpu.sync_copy(zbuf.at[pl.ds(0, sz), :], h_sh.at[pl.ds(r0, sz), :])
        r0 = r0 + sz
    pltpu.sync_copy(zvec_hbm, dzb)
    pltpu.sync_copy(dzb, d_sh.at[pl.ds(s * RPT_D, RPT_D)])
    plsc.subcore_barrier()

    slots = ((src0, dst0, rows0, pv0, semA0, semB0),
             (src1, dst1, rows1, pv1, semA1, semB1),
             (src2, dst2, rows2, pv2, semA2, semB2))
    NSLOT = 3

    def issue(b, slot):
        src_buf, dst_buf, rows_buf, pv_buf, sA, sB = slot
        off = base + b * BATCH
        pltpu.sync_copy(src_hbm.at[pl.ds(off, BATCH)], src_buf)
        pltpu.sync_copy(dst_hbm.at[pl.ds(off, BATCH)], dst_buf)
        pltpu.async_copy(y_hbm.at[src_buf], rows_buf, sA)
        pltpu.async_copy(p_hbm.at[src_buf], pv_buf, sB)

    def drain(slot):
        src_buf, dst_buf, rows_buf, pv_buf, sA, sB = slot
        pltpu.make_async_copy(y_hbm.at[src_buf], rows_buf, sA).wait()
        pltpu.make_async_copy(p_hbm.at[src_buf], pv_buf, sB).wait()
        pltpu.sync_copy(rows_buf, h_sh.at[dst_buf], add=True)
        pltpu.sync_copy(pv_buf, d_sh.at[dst_buf], add=True)

    for k in range(NSLOT - 1):
        issue(k, slots[k])

    def ring(i, carry):
        for k in range(NSLOT):
            b = NSLOT * i + k

            @pl.when(b + NSLOT - 1 < NBATCH)
            def _():
                issue(b + NSLOT - 1, slots[(k + NSLOT - 1) % NSLOT])

            drain(slots[k])
        return carry

    lax.fori_loop(0, NBATCH // NSLOT, ring, 0)
    drain(slots[0])
    drain(slots[1])
    plsc.subcore_barrier()

    r0 = s * RPT_H
    for sz in H_CHUNKS:
        pltpu.sync_copy(h_sh.at[pl.ds(r0, sz), :], zbuf.at[pl.ds(0, sz), :])
        pltpu.sync_copy(zbuf.at[pl.ds(0, sz), :], out_hbm.at[c, pl.ds(r0, sz), :])
        r0 = r0 + sz
    pltpu.sync_copy(d_sh.at[pl.ds(s * RPT_D, RPT_D)], dzb)
    pltpu.sync_copy(dzb, dout_hbm.at[c, pl.ds(s * RPT_D, RPT_D)])


def _sc_scatter(y, p, src, dst):
    mesh = plsc.VectorSubcoreMesh(core_axis_name="c", subcore_axis_name="s")
    zrows = jnp.zeros((128, D), jnp.float32)
    zvec = jnp.zeros((RPT_D,), jnp.float32)
    kern = pl.kernel(
        _sc_body,
        out_type=[
            jax.ShapeDtypeStruct((NC, N_PAD_H, D), jnp.float32),
            jax.ShapeDtypeStruct((NC, N_PAD_D), jnp.float32),
        ],
        mesh=mesh,
        scratch_types=(
            [
                pltpu.VMEM((BATCH,), jnp.int32),
                pltpu.VMEM((BATCH,), jnp.int32),
                pltpu.VMEM((BATCH, D), jnp.float32),
                pltpu.VMEM((BATCH,), jnp.float32),
            ] * 3
            + [
                pltpu.VMEM((128, D), jnp.float32),
                pltpu.VMEM((RPT_D,), jnp.float32),
                pltpu.VMEM_SHARED((N_PAD_H, D), jnp.float32),
                pltpu.VMEM_SHARED((N_PAD_D,), jnp.float32),
            ]
            + [pltpu.SemaphoreType.DMA] * 6
        ),
    )
    return kern(y, p, src, dst, zrows, zvec)


def _finish_body(hp_ref, dp_ref, out_ref):
    hp = hp_ref[...]
    dp = dp_ref[...]
    tot = hp[0] + hp[1]
    den = dp[0] + dp[1]
    inv = jnp.where(den > 0.0, 1.0 / jnp.where(den > 0.0, den, 1.0), 0.0)
    out_ref[...] = tot * inv


def _finish(h_parts, d_parts):
    return pl.pallas_call(
        _finish_body,
        grid=(N_BLKS,),
        in_specs=[
            pl.BlockSpec((NC, R_BLK, D), lambda i: (0, i, 0)),
            pl.BlockSpec((NC, R_BLK, 1), lambda i: (0, i, 0)),
        ],
        out_specs=pl.BlockSpec((R_BLK, D), lambda i: (i, 0)),
        out_shape=jax.ShapeDtypeStruct((N_NODES, D), jnp.float32),
    )(h_parts, d_parts)


@jax.jit
def kernel(x, edge_index, W, a):
    ei = edge_index.astype(jnp.int32)
    src = ei[0]
    dst = ei[1]
    a1 = a[:D].reshape(D, 1)
    y, p2d = _prep(x, W, a1)
    p = p2d.reshape(N_NODES)
    h_parts, d_parts = _sc_scatter(y, p, src, dst)
    return _finish(h_parts, d_parts[:, :, None])


# speedup vs baseline: 31.8557x; 1.0231x over previous
import jax
import jax.numpy as jnp
from jax import lax
from jax.experimental import pallas as pl
from jax.experimental.pallas import tpu as pltpu
from jax.experimental.pallas import tpu_sc as plsc

N_NODES = 10000
N_EDGES = 320000
D = 128

NC = 2
NS = 16
NWORK = NC * NS
EDGES_PER_WORKER = N_EDGES // NWORK
BATCH = 80
NBATCH = EDGES_PER_WORKER // BATCH
N_PAD_H = 10112
RPT_H = N_PAD_H // NS
H_CHUNKS = (128, 128, 128, 128, 120)
N_PAD_D = 10240
RPT_D = N_PAD_D // NS

R_BLK = 2000
N_BLKS = N_NODES // R_BLK


def _prep_body(x_ref, w_ref, a1_ref, y_ref, p_ref, cmax_ref):
    pidx = pl.program_id(0)
    blk = pl.program_id(1)

    @pl.when((pidx == 0) & (blk == 0))
    def _():
        cmax_ref[0] = -jnp.inf

    z = jnp.dot(x_ref[...], w_ref[...], preferred_element_type=jnp.float32)
    s1 = jnp.dot(z, a1_ref[...], preferred_element_type=jnp.float32)

    @pl.when(pidx == 0)
    def _():
        cmax_ref[0] = jnp.maximum(cmax_ref[0], jnp.max(s1))

    p = jnp.exp(s1 - cmax_ref[0])
    y_ref[...] = p * z
    p_ref[...] = p


def _prep(x, W, a1):
    return pl.pallas_call(
        _prep_body,
        grid=(2, N_BLKS),
        in_specs=[
            pl.BlockSpec((R_BLK, D), lambda p, i: (i, 0)),
            pl.BlockSpec((D, D), lambda p, i: (0, 0)),
            pl.BlockSpec((D, 1), lambda p, i: (0, 0)),
        ],
        out_specs=[
            pl.BlockSpec((R_BLK, D), lambda p, i: (i, 0)),
            pl.BlockSpec((R_BLK, 1), lambda p, i: (i, 0)),
        ],
        out_shape=[
            jax.ShapeDtypeStruct((N_NODES, D), jnp.float32),
            jax.ShapeDtypeStruct((N_NODES, 1), jnp.float32),
        ],
        scratch_shapes=[pltpu.SMEM((1,), jnp.float32)],
    )(x, W, a1)


def _sc_body(y_hbm, p_hbm, src_hbm, dst_hbm, zrows_hbm, zvec_hbm,
             out_hbm, dout_hbm,
             src0, dst0, rows0, pv0, src1, dst1, rows1, pv1,
             src2, dst2, rows2, pv2, zbuf, dzb,
             h_sh, d_sh, semA0, semB0, semC0, semA1, semB1, semC1,
             semA2, semB2, semC2):
    c = lax.axis_index("c")
    s = lax.axis_index("s")
    base = (c * NS + s) * EDGES_PER_WORKER

    pltpu.sync_copy(zrows_hbm, zbuf)
    r0 = s * RPT_H
    for sz in H_CHUNKS:
        pltpu.sync_copy(zbuf.at[pl.ds(0, sz), :], h_sh.at[pl.ds(r0, sz), :])
        r0 = r0 + sz
    pltpu.sync_copy(zvec_hbm, dzb)
    pltpu.sync_copy(dzb, d_sh.at[pl.ds(s * RPT_D, RPT_D)])
    plsc.subcore_barrier()

    slots = ((src0, dst0, rows0, pv0, semA0, semB0, semC0),
             (src1, dst1, rows1, pv1, semA1, semB1, semC1),
             (src2, dst2, rows2, pv2, semA2, semB2, semC2))
    NSLOT = 3

    def issue(b, slot):
        src_buf, dst_buf, rows_buf, pv_buf, sA, sB, sC = slot
        off = base + b * BATCH
        pltpu.sync_copy(src_hbm.at[pl.ds(off, BATCH)], src_buf)
        pltpu.sync_copy(dst_hbm.at[pl.ds(off, BATCH)], dst_buf)
        pltpu.async_copy(y_hbm.at[src_buf], rows_buf, sA)
        pltpu.async_copy(p_hbm.at[src_buf], pv_buf, sB)

    def wait_scatter(slot):
        src_buf, dst_buf, rows_buf, pv_buf, sA, sB, sC = slot
        pltpu.make_async_copy(rows_buf, h_sh.at[dst_buf], sC).wait()
        pltpu.make_async_copy(pv_buf, d_sh.at[dst_buf], sC).wait()

    def drain(slot):
        src_buf, dst_buf, rows_buf, pv_buf, sA, sB, sC = slot
        pltpu.make_async_copy(y_hbm.at[src_buf], rows_buf, sA).wait()
        pltpu.make_async_copy(p_hbm.at[src_buf], pv_buf, sB).wait()
        pltpu.async_copy(rows_buf, h_sh.at[dst_buf], sC, add=True)
        pltpu.async_copy(pv_buf, d_sh.at[dst_buf], sC, add=True)

    for k in range(NSLOT - 1):
        issue(k, slots[k])

    def ring(i, carry):
        for k in range(NSLOT):
            b = NSLOT * i + k
            nxt = slots[(k + NSLOT - 1) % NSLOT]

            @pl.when(b >= 1)
            def _():
                wait_scatter(nxt)

            issue(b + NSLOT - 1, nxt)
            drain(slots[k])
        return carry

    lax.fori_loop(0, NBATCH // NSLOT, ring, 0)
    drain(slots[0])
    drain(slots[1])
    for k in range(NSLOT):
        wait_scatter(slots[k])
    plsc.subcore_barrier()

    r0 = s * RPT_H
    for sz in H_CHUNKS:
        pltpu.sync_copy(h_sh.at[pl.ds(r0, sz), :], zbuf.at[pl.ds(0, sz), :])
        pltpu.sync_copy(zbuf.at[pl.ds(0, sz), :], out_hbm.at[c, pl.ds(r0, sz), :])
        r0 = r0 + sz
    pltpu.sync_copy(d_sh.at[pl.ds(s * RPT_D, RPT_D)], dzb)
    pltpu.sync_copy(dzb, dout_hbm.at[c, pl.ds(s * RPT_D, RPT_D)])


def _sc_scatter(y, p, src, dst):
    mesh = plsc.VectorSubcoreMesh(core_axis_name="c", subcore_axis_name="s")
    zrows = jnp.zeros((128, D), jnp.float32)
    zvec = jnp.zeros((RPT_D,), jnp.float32)
    kern = pl.kernel(
        _sc_body,
        out_type=[
            jax.ShapeDtypeStruct((NC, N_PAD_H, D), jnp.float32),
            jax.ShapeDtypeStruct((NC, N_PAD_D), jnp.float32),
        ],
        mesh=mesh,
        scratch_types=(
            [
                pltpu.VMEM((BATCH,), jnp.int32),
                pltpu.VMEM((BATCH,), jnp.int32),
                pltpu.VMEM((BATCH, D), jnp.float32),
                pltpu.VMEM((BATCH,), jnp.float32),
            ] * 3
            + [
                pltpu.VMEM((128, D), jnp.float32),
                pltpu.VMEM((RPT_D,), jnp.float32),
                pltpu.VMEM_SHARED((N_PAD_H, D), jnp.float32),
                pltpu.VMEM_SHARED((N_PAD_D,), jnp.float32),
            ]
            + [pltpu.SemaphoreType.DMA] * 9
        ),
    )
    return kern(y, p, src, dst, zrows, zvec)


def _finish_body(hp_ref, dp_ref, out_ref):
    hp = hp_ref[...]
    dp = dp_ref[...]
    tot = hp[0] + hp[1]
    den = dp[0] + dp[1]
    inv = jnp.where(den > 0.0, 1.0 / jnp.where(den > 0.0, den, 1.0), 0.0)
    out_ref[...] = tot * inv


def _finish(h_parts, d_parts):
    return pl.pallas_call(
        _finish_body,
        grid=(N_BLKS,),
        in_specs=[
            pl.BlockSpec((NC, R_BLK, D), lambda i: (0, i, 0)),
            pl.BlockSpec((NC, R_BLK, 1), lambda i: (0, i, 0)),
        ],
        out_specs=pl.BlockSpec((R_BLK, D), lambda i: (i, 0)),
        out_shape=jax.ShapeDtypeStruct((N_NODES, D), jnp.float32),
    )(h_parts, d_parts)


@jax.jit
def kernel(x, edge_index, W, a):
    ei = edge_index.astype(jnp.int32)
    src = ei[0]
    dst = ei[1]
    a1 = a[:D].reshape(D, 1)
    y, p2d = _prep(x, W, a1)
    p = p2d.reshape(N_NODES)
    h_parts, d_parts = _sc_scatter(y, p, src, dst)
    return _finish(h_parts, d_parts[:, :, None])


# speedup vs baseline: 36.0894x; 1.1329x over previous
import jax
import jax.numpy as jnp
from jax import lax
from jax.experimental import pallas as pl
from jax.experimental.pallas import tpu as pltpu
from jax.experimental.pallas import tpu_sc as plsc

N_NODES = 10000
N_EDGES = 320000
D = 128

NC = 2
NS = 16
NWORK = NC * NS
EDGES_PER_WORKER = N_EDGES // NWORK
BATCH = 120
NB_FULL = EDGES_PER_WORKER // BATCH
TAIL = EDGES_PER_WORKER - NB_FULL * BATCH
RPT_H = 624
H_CHUNKS_LO = (128, 128, 128, 128, 112)
H_CHUNKS_HI = (128, 128, 128, 128, 128)
N_PAD_D = 10240
RPT_D = N_PAD_D // NS

R_BLK = 2000
N_BLKS = N_NODES // R_BLK


def _prep_body(x_ref, w_ref, a1_ref, y_ref, p_ref, cmax_ref):
    pidx = pl.program_id(0)
    blk = pl.program_id(1)

    @pl.when((pidx == 0) & (blk == 0))
    def _():
        cmax_ref[0] = -jnp.inf

    z = jnp.dot(x_ref[...], w_ref[...], preferred_element_type=jnp.float32)
    s1 = jnp.dot(z, a1_ref[...], preferred_element_type=jnp.float32)

    @pl.when(pidx == 0)
    def _():
        cmax_ref[0] = jnp.maximum(cmax_ref[0], jnp.max(s1))

    p = jnp.exp(s1 - cmax_ref[0])
    y_ref[...] = p * z
    p_ref[...] = p


def _prep(x, W, a1):
    return pl.pallas_call(
        _prep_body,
        grid=(2, N_BLKS),
        in_specs=[
            pl.BlockSpec((R_BLK, D), lambda p, i: (i, 0)),
            pl.BlockSpec((D, D), lambda p, i: (0, 0)),
            pl.BlockSpec((D, 1), lambda p, i: (0, 0)),
        ],
        out_specs=[
            pl.BlockSpec((R_BLK, D), lambda p, i: (i, 0)),
            pl.BlockSpec((R_BLK, 1), lambda p, i: (i, 0)),
        ],
        out_shape=[
            jax.ShapeDtypeStruct((N_NODES, D), jnp.float32),
            jax.ShapeDtypeStruct((N_NODES, 1), jnp.float32),
        ],
        scratch_shapes=[pltpu.SMEM((1,), jnp.float32)],
    )(x, W, a1)


def _sc_body(y_hbm, p_hbm, src_hbm, dst_hbm, zrows_hbm, zvec_hbm,
             out_hbm, dout_hbm,
             src_tail, dst_tail,
             srcb0, dst0, rows0, pv0, srcb1, dst1, rows1, pv1,
             zbuf, dzb,
             h_sh, d_sh, semA0, semB0, semC0, semA1, semB1, semC1):
    c = lax.axis_index("c")
    s = lax.axis_index("s")
    base = (c * NS + s) * EDGES_PER_WORKER

    pltpu.sync_copy(zrows_hbm, zbuf)

    def h_ranges(fn, chunks):
        r0 = s * RPT_H
        for sz in chunks:
            fn(r0, sz)
            r0 = r0 + sz

    @pl.when(s < NS - 1)
    def _():
        h_ranges(lambda r0, sz: pltpu.sync_copy(
            zbuf.at[pl.ds(0, sz), :], h_sh.at[pl.ds(r0, sz), :]), H_CHUNKS_LO)

    @pl.when(s == NS - 1)
    def _():
        h_ranges(lambda r0, sz: pltpu.sync_copy(
            zbuf.at[pl.ds(0, sz), :], h_sh.at[pl.ds(r0, sz), :]), H_CHUNKS_HI)

    pltpu.sync_copy(zvec_hbm, dzb)
    pltpu.sync_copy(dzb, d_sh.at[pl.ds(s * RPT_D, RPT_D)])
    plsc.subcore_barrier()

    slots = ((srcb0, dst0, rows0, pv0, semA0, semB0, semC0),
             (srcb1, dst1, rows1, pv1, semA1, semB1, semC1))
    NSLOT = 2

    def issue(b, slot):
        src_buf, dst_buf, rows_buf, pv_buf, sA, sB, sC = slot
        pltpu.sync_copy(src_hbm.at[pl.ds(base + b * BATCH, BATCH)], src_buf)
        pltpu.sync_copy(dst_hbm.at[pl.ds(base + b * BATCH, BATCH)], dst_buf)
        pltpu.async_copy(y_hbm.at[src_buf], rows_buf, sA)
        pltpu.async_copy(p_hbm.at[src_buf], pv_buf, sB)

    def wait_scatter(slot):
        src_buf, dst_buf, rows_buf, pv_buf, sA, sB, sC = slot
        pltpu.make_async_copy(y_hbm.at[pl.ds(0, BATCH), :], rows_buf, sC).wait()
        pltpu.make_async_copy(p_hbm.at[pl.ds(0, BATCH)], pv_buf, sC).wait()

    def drain(b, slot):
        src_buf, dst_buf, rows_buf, pv_buf, sA, sB, sC = slot
        pltpu.make_async_copy(y_hbm.at[pl.ds(0, BATCH), :], rows_buf, sA).wait()
        pltpu.make_async_copy(p_hbm.at[pl.ds(0, BATCH)], pv_buf, sB).wait()
        pltpu.async_copy(rows_buf, h_sh.at[dst_buf], sC, add=True)
        pltpu.async_copy(pv_buf, d_sh.at[dst_buf], sC, add=True)

    for k in range(NSLOT - 1):
        issue(k, slots[k])

    def ring(i, carry):
        for k in range(NSLOT):
            b = NSLOT * i + k
            nxt = slots[(k + NSLOT - 1) % NSLOT]

            @pl.when(b >= 1)
            def _():
                wait_scatter(nxt)

            @pl.when(b + NSLOT - 1 < NB_FULL)
            def _():
                issue(b + NSLOT - 1, nxt)

            drain(b, slots[k])
        return carry

    lax.fori_loop(0, NB_FULL // NSLOT, ring, 0)
    drain(NB_FULL - 1, slots[0])

    wait_scatter(slots[0])
    pltpu.sync_copy(src_hbm.at[pl.ds(base + NB_FULL * BATCH, TAIL)], src_tail)
    pltpu.sync_copy(dst_hbm.at[pl.ds(base + NB_FULL * BATCH, TAIL)], dst_tail)
    pltpu.async_copy(y_hbm.at[src_tail], rows0.at[pl.ds(0, TAIL), :], semA0)
    pltpu.async_copy(p_hbm.at[src_tail], pv0.at[pl.ds(0, TAIL)], semB0)
    pltpu.make_async_copy(y_hbm.at[pl.ds(0, TAIL), :], rows0.at[pl.ds(0, TAIL), :], semA0).wait()
    pltpu.make_async_copy(p_hbm.at[pl.ds(0, TAIL)], pv0.at[pl.ds(0, TAIL)], semB0).wait()
    pltpu.async_copy(rows0.at[pl.ds(0, TAIL), :], h_sh.at[dst_tail], semC0, add=True)
    pltpu.async_copy(pv0.at[pl.ds(0, TAIL)], d_sh.at[dst_tail], semC0, add=True)

    pltpu.make_async_copy(y_hbm.at[pl.ds(0, TAIL), :], rows0.at[pl.ds(0, TAIL), :], semC0).wait()
    pltpu.make_async_copy(p_hbm.at[pl.ds(0, TAIL)], pv0.at[pl.ds(0, TAIL)], semC0).wait()
    for sl in slots[1:]:
        wait_scatter(sl)
    plsc.subcore_barrier()

    def wb(r0, sz):
        pltpu.sync_copy(h_sh.at[pl.ds(r0, sz), :], zbuf.at[pl.ds(0, sz), :])
        pltpu.sync_copy(zbuf.at[pl.ds(0, sz), :], out_hbm.at[c, pl.ds(r0, sz), :])

    @pl.when(s < NS - 1)
    def _():
        h_ranges(wb, H_CHUNKS_LO)

    @pl.when(s == NS - 1)
    def _():
        h_ranges(wb, H_CHUNKS_HI)

    pltpu.sync_copy(d_sh.at[pl.ds(s * RPT_D, RPT_D)], dzb)
    pltpu.sync_copy(dzb, dout_hbm.at[c, pl.ds(s * RPT_D, RPT_D)])


def _sc_scatter(y, p, src, dst):
    mesh = plsc.VectorSubcoreMesh(core_axis_name="c", subcore_axis_name="s")
    zrows = jnp.zeros((128, D), jnp.float32)
    zvec = jnp.zeros((RPT_D,), jnp.float32)
    kern = pl.kernel(
        _sc_body,
        out_type=[
            jax.ShapeDtypeStruct((NC, N_NODES, D), jnp.float32),
            jax.ShapeDtypeStruct((NC, N_PAD_D), jnp.float32),
        ],
        mesh=mesh,
        scratch_types=(
            [
                pltpu.VMEM((TAIL,), jnp.int32),
                pltpu.VMEM((TAIL,), jnp.int32),
            ]
            + [
                pltpu.VMEM((BATCH,), jnp.int32),
                pltpu.VMEM((BATCH,), jnp.int32),
                pltpu.VMEM((BATCH, D), jnp.float32),
                pltpu.VMEM((BATCH,), jnp.float32),
            ] * 2
            + [
                pltpu.VMEM((128, D), jnp.float32),
                pltpu.VMEM((RPT_D,), jnp.float32),
                pltpu.VMEM_SHARED((N_NODES, D), jnp.float32),
                pltpu.VMEM_SHARED((N_PAD_D,), jnp.float32),
            ]
            + [pltpu.SemaphoreType.DMA] * 6
        ),
    )
    return kern(y, p, src, dst, zrows, zvec)


def _finish_body(hp_ref, dp_ref, out_ref):
    hp = hp_ref[...]
    dp = dp_ref[...]
    tot = hp[0] + hp[1]
    den = dp[0] + dp[1]
    inv = jnp.where(den > 0.0, 1.0 / jnp.where(den > 0.0, den, 1.0), 0.0)
    out_ref[...] = tot * inv


def _finish(h_parts, d_parts):
    return pl.pallas_call(
        _finish_body,
        grid=(N_BLKS,),
        in_specs=[
            pl.BlockSpec((NC, R_BLK, D), lambda i: (0, i, 0)),
            pl.BlockSpec((NC, R_BLK, 1), lambda i: (0, i, 0)),
        ],
        out_specs=pl.BlockSpec((R_BLK, D), lambda i: (i, 0)),
        out_shape=jax.ShapeDtypeStruct((N_NODES, D), jnp.float32),
    )(h_parts, d_parts)


@jax.jit
def kernel(x, edge_index, W, a):
    ei = edge_index.astype(jnp.int32)
    src = ei[0]
    dst = ei[1]
    a1 = a[:D].reshape(D, 1)
    y, p2d = _prep(x, W, a1)
    p = p2d.reshape(N_NODES)
    h_parts, d_parts = _sc_scatter(y, p, src, dst)
    return _finish(h_parts, d_parts[:, :, None])


# speedup vs baseline: 40.5546x; 1.1237x over previous
import jax
import jax.numpy as jnp
from jax import lax
from jax.experimental import pallas as pl
from jax.experimental.pallas import tpu as pltpu
from jax.experimental.pallas import tpu_sc as plsc

N_NODES = 10000
N_EDGES = 320000
D = 128

NC = 2
NS = 16
NWORK = NC * NS
EDGES_PER_WORKER = N_EDGES // NWORK
BATCH = 112
NB_FULL = EDGES_PER_WORKER // BATCH
TAIL = EDGES_PER_WORKER - NB_FULL * BATCH
PKW = 2 * EDGES_PER_WORKER
RPT_H = 624
H_CHUNKS_LO = (128, 128, 128, 128, 112)
H_CHUNKS_HI = (128, 128, 128, 128, 128)
N_PAD_D = 10240
RPT_D = N_PAD_D // NS

R_BLK = 2000
N_BLKS = N_NODES // R_BLK


def _prep_body(x_ref, w_ref, a1_ref, y_ref, p_ref, cmax_ref):
    pidx = pl.program_id(0)
    blk = pl.program_id(1)

    @pl.when((pidx == 0) & (blk == 0))
    def _():
        cmax_ref[0] = -jnp.inf

    z = jnp.dot(x_ref[...], w_ref[...], preferred_element_type=jnp.float32)
    s1 = jnp.dot(z, a1_ref[...], preferred_element_type=jnp.float32)

    @pl.when(pidx == 0)
    def _():
        cmax_ref[0] = jnp.maximum(cmax_ref[0], jnp.max(s1))

    p = jnp.exp(s1 - cmax_ref[0])
    y_ref[...] = p * z
    p_ref[...] = p


def _prep(x, W, a1):
    return pl.pallas_call(
        _prep_body,
        grid=(2, N_BLKS),
        in_specs=[
            pl.BlockSpec((R_BLK, D), lambda p, i: (i, 0)),
            pl.BlockSpec((D, D), lambda p, i: (0, 0)),
            pl.BlockSpec((D, 1), lambda p, i: (0, 0)),
        ],
        out_specs=[
            pl.BlockSpec((R_BLK, D), lambda p, i: (i, 0)),
            pl.BlockSpec((R_BLK, 1), lambda p, i: (i, 0)),
        ],
        out_shape=[
            jax.ShapeDtypeStruct((N_NODES, D), jnp.float32),
            jax.ShapeDtypeStruct((N_NODES, 1), jnp.float32),
        ],
        scratch_shapes=[pltpu.SMEM((1,), jnp.float32)],
    )(x, W, a1)


def _sc_body(y_hbm, p_hbm, pk_hbm, zrows_hbm, zvec_hbm,
             out_hbm, dout_hbm,
             src_tail, dst_tail, tailbuf,
             pkb0, srcb0, dst0, rows0, pv0, pkb1, srcb1, dst1, rows1, pv1,
             zbuf, dzb,
             h_sh, d_sh, semP0, semA0, semB0, semC0, semP1, semA1, semB1, semC1):
    c = lax.axis_index("c")
    s = lax.axis_index("s")
    pkbase = (c * NS + s) * PKW

    pltpu.sync_copy(zrows_hbm, zbuf)

    def h_ranges(fn, chunks):
        r0 = s * RPT_H
        for sz in chunks:
            fn(r0, sz)
            r0 = r0 + sz

    @pl.when(s < NS - 1)
    def _():
        h_ranges(lambda r0, sz: pltpu.sync_copy(
            zbuf.at[pl.ds(0, sz), :], h_sh.at[pl.ds(r0, sz), :]), H_CHUNKS_LO)

    @pl.when(s == NS - 1)
    def _():
        h_ranges(lambda r0, sz: pltpu.sync_copy(
            zbuf.at[pl.ds(0, sz), :], h_sh.at[pl.ds(r0, sz), :]), H_CHUNKS_HI)

    pltpu.sync_copy(zvec_hbm, dzb)
    pltpu.sync_copy(dzb, d_sh.at[pl.ds(s * RPT_D, RPT_D)])
    plsc.subcore_barrier()

    slots = ((pkb0, srcb0, dst0, rows0, pv0, semP0, semA0, semB0, semC0),
             (pkb1, srcb1, dst1, rows1, pv1, semP1, semA1, semB1, semC1))

    def load_pk(b, slot):
        pkb = slot[0]
        pltpu.async_copy(pk_hbm.at[pl.ds(pkbase + b * 2 * BATCH, 2 * BATCH)],
                         pkb, slot[5])

    def wait_pk(slot):
        pltpu.make_async_copy(pk_hbm.at[pl.ds(0, 2 * BATCH)], slot[0], slot[5]).wait()

    def moves(slot):
        pkb, src_buf, dst_buf = slot[0], slot[1], slot[2]
        for g in range(BATCH // 16):
            src_buf[pl.ds(g * 16, 16)] = pkb[pl.ds(g * 16, 16)]
        for g in range(BATCH // 16):
            dst_buf[pl.ds(g * 16, 16)] = pkb[pl.ds(BATCH + g * 16, 16)]

    def fire_gathers(slot):
        pltpu.async_copy(y_hbm.at[slot[1]], slot[3], slot[6])
        pltpu.async_copy(p_hbm.at[slot[1]], slot[4], slot[7])

    def wait_scatter(slot):
        pltpu.make_async_copy(y_hbm.at[pl.ds(0, BATCH), :], slot[3], slot[8]).wait()
        pltpu.make_async_copy(p_hbm.at[pl.ds(0, BATCH)], slot[4], slot[8]).wait()

    def drain(slot):
        pltpu.make_async_copy(y_hbm.at[pl.ds(0, BATCH), :], slot[3], slot[6]).wait()
        pltpu.make_async_copy(p_hbm.at[pl.ds(0, BATCH)], slot[4], slot[7]).wait()
        pltpu.async_copy(slot[3], h_sh.at[slot[2]], slot[8], add=True)
        pltpu.async_copy(slot[4], d_sh.at[slot[2]], slot[8], add=True)

    load_pk(0, slots[0])
    load_pk(1, slots[1])
    wait_pk(slots[0])
    moves(slots[0])
    fire_gathers(slots[0])

    def ring(i, carry):
        for k in range(2):
            b = 2 * i + k
            nxt = slots[(k + 1) % 2]

            @pl.when(b >= 1)
            def _():
                wait_scatter(nxt)

            wait_pk(nxt)
            moves(nxt)
            fire_gathers(nxt)

            @pl.when(b + 2 < NB_FULL)
            def _():
                load_pk(b + 2, slots[k])

            drain(slots[k])
        return carry

    lax.fori_loop(0, NB_FULL // 2, ring, 0)
    wait_scatter(slots[1])
    drain(slots[0])

    pltpu.sync_copy(pk_hbm.at[pl.ds(pkbase + NB_FULL * 2 * BATCH, 2 * TAIL)], tailbuf)
    for g in range(TAIL // 16):
        src_tail[pl.ds(g * 16, 16)] = tailbuf[pl.ds(g * 16, 16)]
    for g in range(TAIL // 16):
        dst_tail[pl.ds(g * 16, 16)] = tailbuf[pl.ds(TAIL + g * 16, 16)]
    pltpu.async_copy(y_hbm.at[src_tail], rows1.at[pl.ds(0, TAIL), :], semA1)
    pltpu.async_copy(p_hbm.at[src_tail], pv1.at[pl.ds(0, TAIL)], semB1)
    pltpu.make_async_copy(y_hbm.at[pl.ds(0, TAIL), :], rows1.at[pl.ds(0, TAIL), :], semA1).wait()
    pltpu.make_async_copy(p_hbm.at[pl.ds(0, TAIL)], pv1.at[pl.ds(0, TAIL)], semB1).wait()
    pltpu.async_copy(rows1.at[pl.ds(0, TAIL), :], h_sh.at[dst_tail], semC1, add=True)
    pltpu.async_copy(pv1.at[pl.ds(0, TAIL)], d_sh.at[dst_tail], semC1, add=True)

    pltpu.make_async_copy(y_hbm.at[pl.ds(0, TAIL), :], rows1.at[pl.ds(0, TAIL), :], semC1).wait()
    pltpu.make_async_copy(p_hbm.at[pl.ds(0, TAIL)], pv1.at[pl.ds(0, TAIL)], semC1).wait()
    wait_scatter(slots[0])
    plsc.subcore_barrier()

    def wb(r0, sz):
        pltpu.sync_copy(h_sh.at[pl.ds(r0, sz), :], zbuf.at[pl.ds(0, sz), :])
        pltpu.sync_copy(zbuf.at[pl.ds(0, sz), :], out_hbm.at[c, pl.ds(r0, sz), :])

    @pl.when(s < NS - 1)
    def _():
        h_ranges(wb, H_CHUNKS_LO)

    @pl.when(s == NS - 1)
    def _():
        h_ranges(wb, H_CHUNKS_HI)

    pltpu.sync_copy(d_sh.at[pl.ds(s * RPT_D, RPT_D)], dzb)
    pltpu.sync_copy(dzb, dout_hbm.at[c, pl.ds(s * RPT_D, RPT_D)])


def _sc_scatter(y, p, packed):
    mesh = plsc.VectorSubcoreMesh(core_axis_name="c", subcore_axis_name="s")
    zrows = jnp.zeros((128, D), jnp.float32)
    zvec = jnp.zeros((RPT_D,), jnp.float32)
    kern = pl.kernel(
        _sc_body,
        out_type=[
            jax.ShapeDtypeStruct((NC, N_NODES, D), jnp.float32),
            jax.ShapeDtypeStruct((NC, N_PAD_D), jnp.float32),
        ],
        mesh=mesh,
        scratch_types=(
            [
                pltpu.VMEM((TAIL,), jnp.int32),
                pltpu.VMEM((TAIL,), jnp.int32),
                pltpu.VMEM((2 * TAIL,), jnp.int32),
            ]
            + [
                pltpu.VMEM((2 * BATCH,), jnp.int32),
                pltpu.VMEM((BATCH,), jnp.int32),
                pltpu.VMEM((BATCH,), jnp.int32),
                pltpu.VMEM((BATCH, D), jnp.float32),
                pltpu.VMEM((BATCH,), jnp.float32),
            ] * 2
            + [
                pltpu.VMEM((128, D), jnp.float32),
                pltpu.VMEM((RPT_D,), jnp.float32),
                pltpu.VMEM_SHARED((N_NODES, D), jnp.float32),
                pltpu.VMEM_SHARED((N_PAD_D,), jnp.float32),
            ]
            + [pltpu.SemaphoreType.DMA] * 8
        ),
    )
    return kern(y, p, packed, zrows, zvec)


def _finish_body(hp_ref, dp_ref, out_ref):
    hp = hp_ref[...]
    dp = dp_ref[...]
    tot = hp[0] + hp[1]
    den = dp[0] + dp[1]
    inv = jnp.where(den > 0.0, 1.0 / jnp.where(den > 0.0, den, 1.0), 0.0)
    out_ref[...] = tot * inv


def _finish(h_parts, d_parts):
    return pl.pallas_call(
        _finish_body,
        grid=(N_BLKS,),
        in_specs=[
            pl.BlockSpec((NC, R_BLK, D), lambda i: (0, i, 0)),
            pl.BlockSpec((NC, R_BLK, 1), lambda i: (0, i, 0)),
        ],
        out_specs=pl.BlockSpec((R_BLK, D), lambda i: (i, 0)),
        out_shape=jax.ShapeDtypeStruct((N_NODES, D), jnp.float32),
    )(h_parts, d_parts)


@jax.jit
def kernel(x, edge_index, W, a):
    ei = edge_index.astype(jnp.int32)
    src = ei[0]
    dst = ei[1]
    a1 = a[:D].reshape(D, 1)
    y, p2d = _prep(x, W, a1)
    p = p2d.reshape(N_NODES)
    m = NB_FULL * BATCH
    sp = src.reshape(NWORK, EDGES_PER_WORKER)
    dp = dst.reshape(NWORK, EDGES_PER_WORKER)
    main = jnp.concatenate(
        [sp[:, :m].reshape(NWORK, NB_FULL, BATCH),
         dp[:, :m].reshape(NWORK, NB_FULL, BATCH)], axis=2)
    tails = jnp.concatenate([sp[:, m:], dp[:, m:]], axis=1)
    packed = jnp.concatenate(
        [main.reshape(NWORK, NB_FULL * 2 * BATCH), tails], axis=1).reshape(-1)
    h_parts, d_parts = _sc_scatter(y, p, packed)
    return _finish(h_parts, d_parts[:, :, None])


# speedup vs baseline: 43.7397x; 1.0785x over previous
import jax
import jax.numpy as jnp
from jax import lax
from jax.experimental import pallas as pl
from jax.experimental.pallas import tpu as pltpu
from jax.experimental.pallas import tpu_sc as plsc

N_NODES = 10000
N_EDGES = 320000
D = 128

NC = 2
NS = 16
NWORK = NC * NS
EDGES_PER_WORKER = N_EDGES // NWORK
BATCH = 112
NB_FULL = EDGES_PER_WORKER // BATCH
TAIL = EDGES_PER_WORKER - NB_FULL * BATCH
RPT_H = 624
H_CHUNKS_LO = (128, 128, 128, 128, 112)
H_CHUNKS_HI = (128, 128, 128, 128, 128)
N_PAD_D = 10240
RPT_D = N_PAD_D // NS

R_BLK = 2000
N_BLKS = N_NODES // R_BLK


def _prep_body(x_ref, w_ref, a1_ref, y_ref, p_ref, cmax_ref):
    pidx = pl.program_id(0)
    blk = pl.program_id(1)

    @pl.when((pidx == 0) & (blk == 0))
    def _():
        cmax_ref[0] = -jnp.inf

    z = jnp.dot(x_ref[...], w_ref[...], preferred_element_type=jnp.float32)
    s1 = jnp.dot(z, a1_ref[...], preferred_element_type=jnp.float32)

    @pl.when(pidx == 0)
    def _():
        cmax_ref[0] = jnp.maximum(cmax_ref[0], jnp.max(s1))

    p = jnp.exp(s1 - cmax_ref[0])
    y_ref[...] = p * z
    p_ref[...] = p


def _prep(x, W, a1):
    return pl.pallas_call(
        _prep_body,
        grid=(2, N_BLKS),
        in_specs=[
            pl.BlockSpec((R_BLK, D), lambda p, i: (i, 0)),
            pl.BlockSpec((D, D), lambda p, i: (0, 0)),
            pl.BlockSpec((D, 1), lambda p, i: (0, 0)),
        ],
        out_specs=[
            pl.BlockSpec((R_BLK, D), lambda p, i: (i, 0)),
            pl.BlockSpec((R_BLK, 1), lambda p, i: (i, 0)),
        ],
        out_shape=[
            jax.ShapeDtypeStruct((N_NODES, D), jnp.float32),
            jax.ShapeDtypeStruct((N_NODES, 1), jnp.float32),
        ],
        scratch_shapes=[pltpu.SMEM((1,), jnp.float32)],
    )(x, W, a1)


def _sc_body(y_hbm, p_hbm, src_hbm, dst_hbm, zrows_hbm, zvec_hbm,
             out_hbm, dout_hbm,
             src_tail, dst_tail, tailbuf,
             pkb0, srcb0, dst0, rows0, pv0, pkb1, srcb1, dst1, rows1, pv1,
             zbuf, dzb,
             h_sh, d_sh, semP0, semA0, semB0, semC0, semP1, semA1, semB1, semC1):
    c = lax.axis_index("c")
    s = lax.axis_index("s")
    base = (c * NS + s) * EDGES_PER_WORKER

    pltpu.sync_copy(zrows_hbm, zbuf)

    def h_ranges(fn, chunks):
        r0 = s * RPT_H
        for sz in chunks:
            fn(r0, sz)
            r0 = r0 + sz

    @pl.when(s < NS - 1)
    def _():
        h_ranges(lambda r0, sz: pltpu.sync_copy(
            zbuf.at[pl.ds(0, sz), :], h_sh.at[pl.ds(r0, sz), :]), H_CHUNKS_LO)

    @pl.when(s == NS - 1)
    def _():
        h_ranges(lambda r0, sz: pltpu.sync_copy(
            zbuf.at[pl.ds(0, sz), :], h_sh.at[pl.ds(r0, sz), :]), H_CHUNKS_HI)

    pltpu.sync_copy(zvec_hbm, dzb)
    pltpu.sync_copy(dzb, d_sh.at[pl.ds(s * RPT_D, RPT_D)])
    plsc.subcore_barrier()

    slots = ((pkb0, srcb0, dst0, rows0, pv0, semP0, semA0, semB0, semC0),
             (pkb1, srcb1, dst1, rows1, pv1, semP1, semA1, semB1, semC1))

    def load_pk(b, slot):
        pkb = slot[0]
        off = base + b * BATCH
        pltpu.async_copy(src_hbm.at[pl.ds(off, BATCH)], pkb.at[pl.ds(0, BATCH)], slot[5])
        pltpu.async_copy(dst_hbm.at[pl.ds(off, BATCH)], pkb.at[pl.ds(BATCH, BATCH)], slot[5])

    def wait_pk(slot):
        pltpu.make_async_copy(src_hbm.at[pl.ds(0, BATCH)], slot[0].at[pl.ds(0, BATCH)], slot[5]).wait()
        pltpu.make_async_copy(src_hbm.at[pl.ds(0, BATCH)], slot[0].at[pl.ds(BATCH, BATCH)], slot[5]).wait()

    def moves(slot):
        pkb, src_buf, dst_buf = slot[0], slot[1], slot[2]
        for g in range(BATCH // 16):
            src_buf[pl.ds(g * 16, 16)] = pkb[pl.ds(g * 16, 16)]
        for g in range(BATCH // 16):
            dst_buf[pl.ds(g * 16, 16)] = pkb[pl.ds(BATCH + g * 16, 16)]

    def fire_gathers(slot):
        pltpu.async_copy(y_hbm.at[slot[1]], slot[3], slot[6])
        pltpu.async_copy(p_hbm.at[slot[1]], slot[4], slot[7])

    def wait_scatter(slot):
        pltpu.make_async_copy(y_hbm.at[pl.ds(0, BATCH), :], slot[3], slot[8]).wait()
        pltpu.make_async_copy(p_hbm.at[pl.ds(0, BATCH)], slot[4], slot[8]).wait()

    def drain(slot):
        pltpu.make_async_copy(y_hbm.at[pl.ds(0, BATCH), :], slot[3], slot[6]).wait()
        pltpu.make_async_copy(p_hbm.at[pl.ds(0, BATCH)], slot[4], slot[7]).wait()
        pltpu.async_copy(slot[3], h_sh.at[slot[2]], slot[8], add=True)
        pltpu.async_copy(slot[4], d_sh.at[slot[2]], slot[8], add=True)

    load_pk(0, slots[0])
    load_pk(1, slots[1])
    wait_pk(slots[0])
    moves(slots[0])
    fire_gathers(slots[0])

    def ring(i, carry):
        for k in range(2):
            b = 2 * i + k
            nxt = slots[(k + 1) % 2]

            @pl.when(b >= 1)
            def _():
                wait_scatter(nxt)

            wait_pk(nxt)
            moves(nxt)
            fire_gathers(nxt)

            @pl.when(b + 2 < NB_FULL)
            def _():
                load_pk(b + 2, slots[k])

            drain(slots[k])
        return carry

    lax.fori_loop(0, NB_FULL // 2, ring, 0)
    wait_scatter(slots[1])
    drain(slots[0])

    toff = base + NB_FULL * BATCH
    pltpu.sync_copy(src_hbm.at[pl.ds(toff, TAIL)], tailbuf.at[pl.ds(0, TAIL)])
    pltpu.sync_copy(dst_hbm.at[pl.ds(toff, TAIL)], tailbuf.at[pl.ds(TAIL, TAIL)])
    for g in range(TAIL // 16):
        src_tail[pl.ds(g * 16, 16)] = tailbuf[pl.ds(g * 16, 16)]
    for g in range(TAIL // 16):
        dst_tail[pl.ds(g * 16, 16)] = tailbuf[pl.ds(TAIL + g * 16, 16)]
    pltpu.async_copy(y_hbm.at[src_tail], rows1.at[pl.ds(0, TAIL), :], semA1)
    pltpu.async_copy(p_hbm.at[src_tail], pv1.at[pl.ds(0, TAIL)], semB1)
    pltpu.make_async_copy(y_hbm.at[pl.ds(0, TAIL), :], rows1.at[pl.ds(0, TAIL), :], semA1).wait()
    pltpu.make_async_copy(p_hbm.at[pl.ds(0, TAIL)], pv1.at[pl.ds(0, TAIL)], semB1).wait()
    pltpu.async_copy(rows1.at[pl.ds(0, TAIL), :], h_sh.at[dst_tail], semC1, add=True)
    pltpu.async_copy(pv1.at[pl.ds(0, TAIL)], d_sh.at[dst_tail], semC1, add=True)

    pltpu.make_async_copy(y_hbm.at[pl.ds(0, TAIL), :], rows1.at[pl.ds(0, TAIL), :], semC1).wait()
    pltpu.make_async_copy(p_hbm.at[pl.ds(0, TAIL)], pv1.at[pl.ds(0, TAIL)], semC1).wait()
    wait_scatter(slots[0])
    plsc.subcore_barrier()

    def wb(r0, sz):
        pltpu.sync_copy(h_sh.at[pl.ds(r0, sz), :], zbuf.at[pl.ds(0, sz), :])
        pltpu.sync_copy(zbuf.at[pl.ds(0, sz), :], out_hbm.at[c, pl.ds(r0, sz), :])

    @pl.when(s < NS - 1)
    def _():
        h_ranges(wb, H_CHUNKS_LO)

    @pl.when(s == NS - 1)
    def _():
        h_ranges(wb, H_CHUNKS_HI)

    pltpu.sync_copy(d_sh.at[pl.ds(s * RPT_D, RPT_D)], dzb)
    pltpu.sync_copy(dzb, dout_hbm.at[c, pl.ds(s * RPT_D, RPT_D)])


def _sc_scatter(y, p, src, dst):
    mesh = plsc.VectorSubcoreMesh(core_axis_name="c", subcore_axis_name="s")
    zrows = jnp.zeros((128, D), jnp.float32)
    zvec = jnp.zeros((RPT_D,), jnp.float32)
    kern = pl.kernel(
        _sc_body,
        out_type=[
            jax.ShapeDtypeStruct((NC, N_NODES, D), jnp.float32),
            jax.ShapeDtypeStruct((NC, N_PAD_D), jnp.float32),
        ],
        mesh=mesh,
        scratch_types=(
            [
                pltpu.VMEM((TAIL,), jnp.int32),
                pltpu.VMEM((TAIL,), jnp.int32),
                pltpu.VMEM((2 * TAIL,), jnp.int32),
            ]
            + [
                pltpu.VMEM((2 * BATCH,), jnp.int32),
                pltpu.VMEM((BATCH,), jnp.int32),
                pltpu.VMEM((BATCH,), jnp.int32),
                pltpu.VMEM((BATCH, D), jnp.float32),
                pltpu.VMEM((BATCH,), jnp.float32),
            ] * 2
            + [
                pltpu.VMEM((128, D), jnp.float32),
                pltpu.VMEM((RPT_D,), jnp.float32),
                pltpu.VMEM_SHARED((N_NODES, D), jnp.float32),
                pltpu.VMEM_SHARED((N_PAD_D,), jnp.float32),
            ]
            + [pltpu.SemaphoreType.DMA] * 8
        ),
    )
    return kern(y, p, src, dst, zrows, zvec)


def _finish_body(hp_ref, dp_ref, out_ref):
    hp = hp_ref[...]
    dp = dp_ref[...]
    tot = hp[0] + hp[1]
    den = dp[0] + dp[1]
    inv = jnp.where(den > 0.0, 1.0 / jnp.where(den > 0.0, den, 1.0), 0.0)
    out_ref[...] = tot * inv


def _finish(h_parts, d_parts):
    return pl.pallas_call(
        _finish_body,
        grid=(N_BLKS,),
        in_specs=[
            pl.BlockSpec((NC, R_BLK, D), lambda i: (0, i, 0)),
            pl.BlockSpec((NC, R_BLK, 1), lambda i: (0, i, 0)),
        ],
        out_specs=pl.BlockSpec((R_BLK, D), lambda i: (i, 0)),
        out_shape=jax.ShapeDtypeStruct((N_NODES, D), jnp.float32),
    )(h_parts, d_parts)


@jax.jit
def kernel(x, edge_index, W, a):
    ei = edge_index.astype(jnp.int32)
    src = ei[0]
    dst = ei[1]
    a1 = a[:D].reshape(D, 1)
    y, p2d = _prep(x, W, a1)
    p = p2d.reshape(N_NODES)
    h_parts, d_parts = _sc_scatter(y, p, src, dst)
    return _finish(h_parts, d_parts[:, :, None])
